# Initial kernel scaffold; baseline (speedup 1.0000x reference)
import jax
import jax.numpy as jnp
from jax.experimental import pallas as pl


def kernel(x, edge_index, W1, b1, W2, b2):
    raise NotImplementedError("write your pallas kernel here")


# speedup vs baseline: 24.3916x; 24.3916x over previous
import functools

import jax
import jax.numpy as jnp
from jax import lax
from jax.experimental import pallas as pl
from jax.experimental.pallas import tpu as pltpu
from jax.experimental.pallas import tpu_sc as plsc

N = 10000
E = 320000
D_IN = 128
D_HID = 256
D_OUT = 64

NC = 2
NS = 16
NW = NC * NS
EPW = E // NW
CHUNK = 125
NCHUNKS = EPW // CHUNK
NPAD = 10240
STRIPE = NPAD // NS
ZROWS = 64

ROWTILE = 1000
GRID = N // ROWTILE

_MESH = plsc.VectorSubcoreMesh(core_axis_name="c", subcore_axis_name="s")


@functools.partial(
    pl.kernel,
    out_type=jax.ShapeDtypeStruct((NC, NPAD), jnp.float32),
    mesh=_MESH,
    scratch_types=[
        pltpu.VMEM((NCHUNKS, CHUNK), jnp.int32),
        pltpu.VMEM((STRIPE,), jnp.float32),
        pltpu.VMEM((128,), jnp.float32),
        pltpu.VMEM_SHARED((NPAD,), jnp.float32),
    ],
)
def _deg_kernel(dst_hbm, zeros_hbm, ones_hbm, deg_hbm, dstv, zb, onesv, acc):
    cid = lax.axis_index("c")
    sid = lax.axis_index("s")
    wid = cid * NS + sid
    pltpu.sync_copy(zeros_hbm, zb)
    pltpu.sync_copy(zb, acc.at[pl.ds(sid * STRIPE, STRIPE)])
    pltpu.sync_copy(ones_hbm, onesv)
    pltpu.sync_copy(dst_hbm.at[wid], dstv)
    plsc.subcore_barrier()

    def body(c, carry):
        pltpu.sync_copy(onesv.at[pl.ds(0, CHUNK)], acc.at[dstv.at[c]], add=True)
        return carry

    lax.fori_loop(0, NCHUNKS, body, 0)
    plsc.subcore_barrier()
    pltpu.sync_copy(acc.at[pl.ds(sid * STRIPE, STRIPE)], zb)
    pltpu.sync_copy(zb, deg_hbm.at[cid, pl.ds(sid * STRIPE, STRIPE)])


def _make_edge_kernel(D):
    @functools.partial(
        pl.kernel,
        out_type=jax.ShapeDtypeStruct((NC, NPAD, D), jnp.float32),
        mesh=_MESH,
        scratch_types=[
            pltpu.VMEM((NCHUNKS, CHUNK), jnp.int32),
            pltpu.VMEM((NCHUNKS, CHUNK), jnp.int32),
            pltpu.VMEM((CHUNK, D), jnp.float32),
            pltpu.VMEM((ZROWS, D), jnp.float32),
            pltpu.VMEM_SHARED((NPAD, D), jnp.float32),
            pltpu.SemaphoreType.DMA,
        ],
    )
    def edge_kernel(y_hbm, src_hbm, dst_hbm, zeros_hbm, p_hbm,
                    srcv, dstv, buf, zb, acc, sem):
        cid = lax.axis_index("c")
        sid = lax.axis_index("s")
        wid = cid * NS + sid
        pltpu.sync_copy(zeros_hbm, zb)
        for j in range(STRIPE // ZROWS):
            pltpu.sync_copy(zb, acc.at[pl.ds(sid * STRIPE + j * ZROWS, ZROWS)])
        pltpu.sync_copy(src_hbm.at[wid], srcv)
        pltpu.sync_copy(dst_hbm.at[wid], dstv)
        plsc.subcore_barrier()

        def body(c, carry):
            pltpu.async_copy(y_hbm.at[srcv.at[c]], buf, sem).wait()
            pltpu.sync_copy(buf, acc.at[dstv.at[c]], add=True)
            return carry

        lax.fori_loop(0, NCHUNKS, body, 0)
        plsc.subcore_barrier()
        for j in range(STRIPE // ZROWS):
            r0 = sid * STRIPE + j * ZROWS
            pltpu.sync_copy(acc.at[pl.ds(r0, ZROWS)], zb)
            pltpu.sync_copy(zb, p_hbm.at[cid, pl.ds(r0, ZROWS)])

    return edge_kernel


_edge128 = _make_edge_kernel(D_IN)


def _dinv(deg0_ref, deg1_ref):
    deg = deg0_ref[...] + deg1_ref[...] + 1.0
    return lax.rsqrt(deg)


def _prep_body(x_ref, deg0_ref, deg1_ref, y_ref):
    y_ref[...] = x_ref[...] * _dinv(deg0_ref, deg1_ref)


def _mid_body(p0_ref, p1_ref, y_ref, deg0_ref, deg1_ref,
              w1_ref, b1_ref, w2_ref, z_ref):
    dinv = _dinv(deg0_ref, deg1_ref)
    agg = (p0_ref[...] + p1_ref[...] + y_ref[...]) * dinv
    h = jnp.maximum(
        jnp.dot(agg, w1_ref[...], preferred_element_type=jnp.float32)
        + b1_ref[...], 0.0)
    z = jnp.dot(h, w2_ref[...], preferred_element_type=jnp.float32) * dinv
    z_ref[...] = jnp.concatenate([z, jnp.zeros_like(z)], axis=1)


def _fin_body(q0_ref, q1_ref, z_ref, deg0_ref, deg1_ref, b2_ref, out_ref):
    dinv = _dinv(deg0_ref, deg1_ref)
    val = (q0_ref[...] + q1_ref[...] + z_ref[...]) * dinv + b2_ref[...]
    out_ref[...] = jnp.maximum(val, 0.0)[:, :D_OUT]


def _row_spec(d):
    return pl.BlockSpec((ROWTILE, d), lambda i: (i, 0))


def _full_spec(r, c):
    return pl.BlockSpec((r, c), lambda i: (0, 0))


_prep_call = pl.pallas_call(
    _prep_body,
    grid=(GRID,),
    in_specs=[_row_spec(D_IN), _row_spec(1), _row_spec(1)],
    out_specs=_row_spec(D_IN),
    out_shape=jax.ShapeDtypeStruct((N, D_IN), jnp.float32),
)

_mid_call = pl.pallas_call(
    _mid_body,
    grid=(GRID,),
    in_specs=[_row_spec(D_IN), _row_spec(D_IN), _row_spec(D_IN),
              _row_spec(1), _row_spec(1),
              _full_spec(D_IN, D_HID), _full_spec(1, D_HID),
              _full_spec(D_HID, D_OUT)],
    out_specs=_row_spec(D_IN),
    out_shape=jax.ShapeDtypeStruct((N, D_IN), jnp.float32),
)

_fin_call = pl.pallas_call(
    _fin_body,
    grid=(GRID,),
    in_specs=[_row_spec(D_IN), _row_spec(D_IN), _row_spec(D_IN),
              _row_spec(1), _row_spec(1), _full_spec(1, D_IN)],
    out_specs=_row_spec(D_OUT),
    out_shape=jax.ShapeDtypeStruct((N, D_OUT), jnp.float32),
)


def kernel(x, edge_index, W1, b1, W2, b2):
    src_r = edge_index[0].reshape(NW, NCHUNKS, CHUNK)
    dst_r = edge_index[1].reshape(NW, NCHUNKS, CHUNK)

    zeros1 = jnp.zeros((STRIPE,), jnp.float32)
    ones1 = jnp.ones((128,), jnp.float32)
    degp = _deg_kernel(dst_r, zeros1, ones1)
    deg0 = degp[0, :N].reshape(N, 1)
    deg1 = degp[1, :N].reshape(N, 1)

    y = _prep_call(x, deg0, deg1)

    zeros128 = jnp.zeros((ZROWS, D_IN), jnp.float32)
    p = _edge128(y, src_r, dst_r, zeros128)
    z = _mid_call(p[0, :N], p[1, :N], y, deg0, deg1,
                  W1, b1.reshape(1, D_HID), W2)

    q = _edge128(z, src_r, dst_r, zeros128)
    b2p = jnp.concatenate([b2, jnp.zeros((D_IN - D_OUT,), jnp.float32)])
    return _fin_call(q[0, :N], q[1, :N], z, deg0, deg1, b2p.reshape(1, D_IN))


# speedup vs baseline: 29.3028x; 1.2013x over previous
import functools

import jax
import jax.numpy as jnp
from jax import lax
from jax.experimental import pallas as pl
from jax.experimental.pallas import tpu as pltpu
from jax.experimental.pallas import tpu_sc as plsc

N = 10000
E = 320000
D_IN = 128
D_HID = 256
D_OUT = 64

NC = 2
NS = 16
NW = NC * NS
CHUNK = 128
NCHUNKS = 80
EPW = NCHUNKS * CHUNK
EPAD = NW * EPW
NPAD = 10240
STRIPE = NPAD // NS
ZROWS = 16

ROWTILE = 1000
GRID = N // ROWTILE

_MESH = plsc.VectorSubcoreMesh(core_axis_name="c", subcore_axis_name="s")


@functools.partial(
    pl.kernel,
    out_type=jax.ShapeDtypeStruct((NC, NPAD), jnp.float32),
    mesh=_MESH,
    scratch_types=[
        pltpu.VMEM((NCHUNKS, 2, CHUNK), jnp.int32),
        pltpu.VMEM((STRIPE,), jnp.float32),
        pltpu.VMEM((CHUNK,), jnp.float32),
        pltpu.VMEM_SHARED((NPAD,), jnp.float32),
    ],
)
def _deg_kernel(e_hbm, zeros_hbm, ones_hbm, deg_hbm, ev, zb, onesv, acc):
    cid = lax.axis_index("c")
    sid = lax.axis_index("s")
    wid = cid * NS + sid
    pltpu.sync_copy(zeros_hbm, zb)
    pltpu.sync_copy(zb, acc.at[pl.ds(sid * STRIPE, STRIPE)])
    pltpu.sync_copy(ones_hbm, onesv)
    pltpu.sync_copy(e_hbm.at[wid], ev)
    plsc.subcore_barrier()

    def body(c, carry):
        pltpu.sync_copy(onesv, acc.at[ev.at[c, 1]], add=True)
        return carry

    lax.fori_loop(0, NCHUNKS, body, 0)
    plsc.subcore_barrier()
    pltpu.sync_copy(acc.at[pl.ds(sid * STRIPE, STRIPE)], zb)
    pltpu.sync_copy(zb, deg_hbm.at[cid, pl.ds(sid * STRIPE, STRIPE)])


def _make_edge_kernel(D):
    @functools.partial(
        pl.kernel,
        out_type=jax.ShapeDtypeStruct((NC, NPAD, D), jnp.float32),
        mesh=_MESH,
        scratch_types=[
            pltpu.VMEM((2, CHUNK), jnp.int32),
            pltpu.VMEM((2, CHUNK), jnp.int32),
            pltpu.VMEM((CHUNK, D), jnp.float32),
            pltpu.VMEM((CHUNK, D), jnp.float32),
            pltpu.VMEM((ZROWS, D), jnp.float32),
            pltpu.VMEM_SHARED((NPAD, D), jnp.float32),
            pltpu.SemaphoreType.DMA,
            pltpu.SemaphoreType.DMA,
            pltpu.SemaphoreType.DMA,
            pltpu.SemaphoreType.DMA,
        ],
    )
    def edge_kernel(y_hbm, e_hbm, zeros_hbm, p_hbm,
                    ib0, ib1, buf0, buf1, zb, acc,
                    isem0, isem1, rsem0, rsem1):
        cid = lax.axis_index("c")
        sid = lax.axis_index("s")
        wid = cid * NS + sid
        ibs = (ib0, ib1)
        isems = (isem0, isem1)
        bufs = (buf0, buf1)
        rsems = (rsem0, rsem1)
        pltpu.sync_copy(zeros_hbm, zb)
        for j in range(STRIPE // ZROWS):
            pltpu.sync_copy(zb, acc.at[pl.ds(sid * STRIPE + j * ZROWS, ZROWS)])
        plsc.subcore_barrier()

        pltpu.async_copy(e_hbm.at[wid, 0], ib0, isem0)
        pltpu.async_copy(e_hbm.at[wid, 1], ib1, isem1)
        pltpu.make_async_copy(e_hbm.at[wid, 0], ib0, isem0).wait()
        pltpu.async_copy(y_hbm.at[ib0.at[0]], buf0, rsem0)

        def body(i, carry):
            for b in range(2):
                c = i * 2 + b

                @pl.when(c + 1 < NCHUNKS)
                def _():
                    pltpu.make_async_copy(
                        e_hbm.at[wid, c + 1], ibs[1 - b], isems[1 - b]).wait()

                pltpu.make_async_copy(
                    y_hbm.at[ibs[b].at[0]], bufs[b], rsems[b]).wait()

                @pl.when(c + 1 < NCHUNKS)
                def _():
                    pltpu.async_copy(
                        y_hbm.at[ibs[1 - b].at[0]], bufs[1 - b], rsems[1 - b])

                pltpu.sync_copy(bufs[b], acc.at[ibs[b].at[1]], add=True)

                @pl.when(c + 2 < NCHUNKS)
                def _():
                    pltpu.async_copy(
                        e_hbm.at[wid, c + 2], ibs[b], isems[b])
            return carry

        lax.fori_loop(0, NCHUNKS // 2, body, 0)
        plsc.subcore_barrier()
        for j in range(STRIPE // ZROWS):
            r0 = sid * STRIPE + j * ZROWS
            pltpu.sync_copy(acc.at[pl.ds(r0, ZROWS)], zb)
            pltpu.sync_copy(zb, p_hbm.at[cid, pl.ds(r0, ZROWS)])

    return edge_kernel


_edge128 = _make_edge_kernel(D_IN)


def _dinv(deg0_ref, deg1_ref):
    deg = deg0_ref[...] + deg1_ref[...] + 1.0
    return lax.rsqrt(deg)


def _prep_body(x_ref, deg0_ref, deg1_ref, y_ref):
    y_ref[...] = x_ref[...] * _dinv(deg0_ref, deg1_ref)


def _mid_body(p0_ref, p1_ref, y_ref, deg0_ref, deg1_ref,
              w1_ref, b1_ref, w2_ref, z_ref):
    dinv = _dinv(deg0_ref, deg1_ref)
    agg = (p0_ref[...] + p1_ref[...] + y_ref[...]) * dinv
    h = jnp.maximum(
        jnp.dot(agg, w1_ref[...], preferred_element_type=jnp.float32)
        + b1_ref[...], 0.0)
    z = jnp.dot(h, w2_ref[...], preferred_element_type=jnp.float32) * dinv
    z_ref[...] = jnp.concatenate([z, jnp.zeros_like(z)], axis=1)


def _fin_body(q0_ref, q1_ref, z_ref, deg0_ref, deg1_ref, b2_ref, out_ref):
    dinv = _dinv(deg0_ref, deg1_ref)
    val = (q0_ref[...] + q1_ref[...] + z_ref[...]) * dinv + b2_ref[...]
    out_ref[...] = jnp.maximum(val, 0.0)[:, :D_OUT]


def _row_spec(d):
    return pl.BlockSpec((ROWTILE, d), lambda i: (i, 0))


def _full_spec(r, c):
    return pl.BlockSpec((r, c), lambda i: (0, 0))


_prep_call = pl.pallas_call(
    _prep_body,
    grid=(GRID,),
    in_specs=[_row_spec(D_IN), _row_spec(1), _row_spec(1)],
    out_specs=_row_spec(D_IN),
    out_shape=jax.ShapeDtypeStruct((N, D_IN), jnp.float32),
)

_mid_call = pl.pallas_call(
    _mid_body,
    grid=(GRID,),
    in_specs=[_row_spec(D_IN), _row_spec(D_IN), _row_spec(D_IN),
              _row_spec(1), _row_spec(1),
              _full_spec(D_IN, D_HID), _full_spec(1, D_HID),
              _full_spec(D_HID, D_OUT)],
    out_specs=_row_spec(D_IN),
    out_shape=jax.ShapeDtypeStruct((N, D_IN), jnp.float32),
)

_fin_call = pl.pallas_call(
    _fin_body,
    grid=(GRID,),
    in_specs=[_row_spec(D_IN), _row_spec(D_IN), _row_spec(D_IN),
              _row_spec(1), _row_spec(1), _full_spec(1, D_IN)],
    out_specs=_row_spec(D_OUT),
    out_shape=jax.ShapeDtypeStruct((N, D_OUT), jnp.float32),
)


def kernel(x, edge_index, W1, b1, W2, b2):
    npad_e = EPAD - E
    pad_src = (jnp.arange(npad_e, dtype=jnp.int32) * 37) % N
    pad_dst = N + (jnp.arange(npad_e, dtype=jnp.int32) % (NPAD - N))
    src_p = jnp.concatenate([edge_index[0], pad_src])
    dst_p = jnp.concatenate([edge_index[1], pad_dst])
    eiw = (jnp.stack([src_p, dst_p], axis=0)
           .reshape(2, NW, NCHUNKS, CHUNK)
           .transpose(1, 2, 0, 3))

    zeros1 = jnp.zeros((STRIPE,), jnp.float32)
    ones1 = jnp.ones((CHUNK,), jnp.float32)
    degp = _deg_kernel(eiw, zeros1, ones1)
    deg0 = degp[0, :N].reshape(N, 1)
    deg1 = degp[1, :N].reshape(N, 1)

    y = _prep_call(x, deg0, deg1)

    zeros128 = jnp.zeros((ZROWS, D_IN), jnp.float32)
    p = _edge128(y, eiw, zeros128)
    z = _mid_call(p[0, :N], p[1, :N], y, deg0, deg1,
                  W1, b1.reshape(1, D_HID), W2)

    q = _edge128(z, eiw, zeros128)
    b2p = jnp.concatenate([b2, jnp.zeros((D_IN - D_OUT,), jnp.float32)])
    return _fin_call(q[0, :N], q[1, :N], z, deg0, deg1, b2p.reshape(1, D_IN))
